# baseline (speedup 1.0000x reference)
import functools

import jax
import jax.numpy as jnp
from jax import lax
from jax.experimental import pallas as pl
from jax.experimental.pallas import tpu as pltpu
from jax.experimental.pallas import tpu_sc as plsc

BATCH = 16384
VEC = 64
NOUT = 30
NB = 1000000
NP = 100000
HB = 524288
HP = 65536

_NC = 2
_NS = 16
_NW = _NC * _NS
_BPW = BATCH // _NW
_CH = 128


def _pair_body(lo_ref, hi_ref, out_ref):
    eye = (lax.broadcasted_iota(jnp.int32, (VEC, VEC), 0)
           == lax.broadcasted_iota(jnp.int32, (VEC, VEC), 1)).astype(jnp.float32)
    dn = (((0,), (0,)), ((), ()))
    lo_t = lax.dot_general(lo_ref[...], eye, dn,
                           preferred_element_type=jnp.float32)
    hi_t = lax.dot_general(hi_ref[...], eye, dn,
                           preferred_element_type=jnp.float32)
    out_ref[...] = jnp.concatenate([lo_t, hi_t], axis=1)


def _tc_pair(tab_t, half):
    blk = 2048
    grid = (half // blk,)
    off = half // blk
    last = (tab_t.shape[1] + blk - 1) // blk - 1
    return pl.pallas_call(
        _pair_body,
        grid=grid,
        in_specs=[
            pl.BlockSpec((VEC, blk), lambda i: (0, i)),
            pl.BlockSpec((VEC, blk), lambda i: (0, jnp.minimum(i + off, last))),
        ],
        out_specs=pl.BlockSpec((blk, 2 * VEC), lambda i: (i, 0)),
        out_shape=jax.ShapeDtypeStruct((half, 2 * VEC), jnp.float32),
        compiler_params=pltpu.CompilerParams(fuse_transposed_lhs_in_matmul=True),
    )(tab_t, tab_t)


def _sc_gather(batter_idx, pitcher_idx, bpair, ppair):
    mesh = plsc.VectorSubcoreMesh(core_axis_name="c", subcore_axis_name="s")

    @functools.partial(
        pl.kernel,
        mesh=mesh,
        out_type=[
            jax.ShapeDtypeStruct((BATCH, 2 * VEC), jnp.float32),
            jax.ShapeDtypeStruct((BATCH, 2 * VEC), jnp.float32),
        ],
        scratch_types=[
            pltpu.VMEM((_BPW,), jnp.int32),
            pltpu.VMEM((_BPW,), jnp.int32),
            pltpu.VMEM((_BPW, 2 * VEC), jnp.float32),
            pltpu.SemaphoreType.DMA,
        ],
    )
    def k(bidx_hbm, pidx_hbm, btab_hbm, ptab_hbm, bout_hbm, pout_hbm,
          bidx_v, pidx_v, rows_v, sem):
        wid = lax.axis_index("s") * _NC + lax.axis_index("c")
        base = wid * _BPW
        pltpu.sync_copy(bidx_hbm.at[pl.ds(base, _BPW)], bidx_v)
        pltpu.sync_copy(pidx_hbm.at[pl.ds(base, _BPW)], pidx_v)
        for g in range(_BPW // 16):
            sl = pl.ds(g * 16, 16)
            bi = bidx_v[sl]
            bidx_v[sl] = jnp.where(bi >= HB, bi - HB, bi)
            pi = pidx_v[sl]
            pidx_v[sl] = jnp.where(pi >= HP, pi - HP, pi)
        for idx_v, tab_hbm, out_hbm in (
            (bidx_v, btab_hbm, bout_hbm),
            (pidx_v, ptab_hbm, pout_hbm),
        ):
            copies = []
            for c in range(_BPW // _CH):
                sl = pl.ds(c * _CH, _CH)
                copies.append(
                    pltpu.async_copy(tab_hbm.at[idx_v.at[sl]], rows_v.at[sl], sem))
            for cp in copies:
                cp.wait()
            pltpu.sync_copy(rows_v, out_hbm.at[pl.ds(base, _BPW)])

    return k(batter_idx, pitcher_idx, bpair, ppair)


def _tc_body(bsel_ref, psel_ref, bpar_ref, ppar_ref, w1_ref, w2_ref, bias_ref,
             out_ref):
    bsel = bsel_ref[...]
    psel = psel_ref[...]
    bemb = jnp.where(bpar_ref[...] > 0.5, bsel[:, VEC:], bsel[:, :VEC])
    pemb = jnp.where(ppar_ref[...] > 0.5, psel[:, VEC:], psel[:, :VEC])
    sb = jax.nn.sigmoid(bemb)
    sp = jax.nn.sigmoid(pemb)
    acc = jnp.dot(sb, w1_ref[...], preferred_element_type=jnp.float32)
    acc += jnp.dot(sp, w2_ref[...], preferred_element_type=jnp.float32)
    out_ref[...] = acc + bias_ref[...]


def _tc_dense(bsel, psel, bpar, ppar, fc_w, fc_b):
    w1 = fc_w[:, :VEC].T
    w2 = fc_w[:, VEC:].T
    bias = fc_b.reshape(1, NOUT)
    blk = 2048
    grid = (BATCH // blk,)
    return pl.pallas_call(
        _tc_body,
        grid=grid,
        in_specs=[
            pl.BlockSpec((blk, 2 * VEC), lambda i: (i, 0)),
            pl.BlockSpec((blk, 2 * VEC), lambda i: (i, 0)),
            pl.BlockSpec((blk, 1), lambda i: (i, 0)),
            pl.BlockSpec((blk, 1), lambda i: (i, 0)),
            pl.BlockSpec((VEC, NOUT), lambda i: (0, 0)),
            pl.BlockSpec((VEC, NOUT), lambda i: (0, 0)),
            pl.BlockSpec((1, NOUT), lambda i: (0, 0)),
        ],
        out_specs=pl.BlockSpec((blk, NOUT), lambda i: (i, 0)),
        out_shape=jax.ShapeDtypeStruct((BATCH, NOUT), jnp.float32),
    )(bsel, psel, bpar, ppar, w1, w2, bias)


@jax.jit
def kernel(batter_idx, pitcher_idx, batter_table, pitcher_table, fc_w, fc_b):
    bpair = _tc_pair(batter_table.T, HB)
    ppair = _tc_pair(pitcher_table.T, HP)
    bsel, psel = _sc_gather(batter_idx, pitcher_idx, bpair, ppair)
    bpar = (batter_idx >= HB).astype(jnp.float32).reshape(-1, 1)
    ppar = (pitcher_idx >= HP).astype(jnp.float32).reshape(-1, 1)
    return _tc_dense(bsel, psel, bpar, ppar, fc_w, fc_b)


# speedup vs baseline: 1.5073x; 1.5073x over previous
import functools

import jax
import jax.numpy as jnp
from jax import lax
from jax.experimental import pallas as pl
from jax.experimental.pallas import tpu as pltpu
from jax.experimental.pallas import tpu_sc as plsc

BATCH = 16384
VEC = 64
NOUT = 30
NQ = 32
HB = 262144
HP = 32768

_NC = 2
_NS = 16
_NW = _NC * _NS
_BPW = BATCH // _NW
_CH = 128


def _quad_body(x0_ref, x1_ref, x2_ref, x3_ref, wt_ref, out_ref):
    dn = (((1,), (0,)), ((), ()))
    wt = wt_ref[...]
    ys = []
    for x_ref in (x0_ref, x1_ref, x2_ref, x3_ref):
        s = jax.nn.sigmoid(x_ref[...])
        ys.append(lax.dot_general(wt, s, dn, preferred_element_type=jnp.float32))
    out_ref[...] = jnp.concatenate(ys, axis=0).T


def _tc_quad(tab_t, half, w_t):
    blk = 2048
    grid = (half // blk,)
    off = half // blk
    last = (tab_t.shape[1] + blk - 1) // blk - 1
    mk = lambda q: (lambda i: (0, jnp.minimum(i + q * off, last)))
    return pl.pallas_call(
        _quad_body,
        grid=grid,
        in_specs=[
            pl.BlockSpec((VEC, blk), mk(0)),
            pl.BlockSpec((VEC, blk), mk(1)),
            pl.BlockSpec((VEC, blk), mk(2)),
            pl.BlockSpec((VEC, blk), mk(3)),
            pl.BlockSpec((NQ, VEC), lambda i: (0, 0)),
        ],
        out_specs=pl.BlockSpec((blk, 4 * NQ), lambda i: (i, 0)),
        out_shape=jax.ShapeDtypeStruct((half, 4 * NQ), jnp.float32),
        compiler_params=pltpu.CompilerParams(fuse_transposed_lhs_in_matmul=True),
    )(tab_t, tab_t, tab_t, tab_t, w_t)


def _sc_gather(batter_idx, pitcher_idx, bquad, pquad):
    mesh = plsc.VectorSubcoreMesh(core_axis_name="c", subcore_axis_name="s")

    @functools.partial(
        pl.kernel,
        mesh=mesh,
        out_type=[
            jax.ShapeDtypeStruct((BATCH, 4 * NQ), jnp.float32),
            jax.ShapeDtypeStruct((BATCH, 4 * NQ), jnp.float32),
        ],
        scratch_types=[
            pltpu.VMEM((_BPW,), jnp.int32),
            pltpu.VMEM((_BPW,), jnp.int32),
            pltpu.VMEM((_BPW, 4 * NQ), jnp.float32),
            pltpu.SemaphoreType.DMA,
        ],
    )
    def k(bidx_hbm, pidx_hbm, btab_hbm, ptab_hbm, bout_hbm, pout_hbm,
          bidx_v, pidx_v, rows_v, sem):
        wid = lax.axis_index("s") * _NC + lax.axis_index("c")
        base = wid * _BPW
        pltpu.sync_copy(bidx_hbm.at[pl.ds(base, _BPW)], bidx_v)
        pltpu.sync_copy(pidx_hbm.at[pl.ds(base, _BPW)], pidx_v)
        for g in range(_BPW // 16):
            sl = pl.ds(g * 16, 16)
            bidx_v[sl] = lax.bitwise_and(bidx_v[sl], HB - 1)
            pidx_v[sl] = lax.bitwise_and(pidx_v[sl], HP - 1)
        for idx_v, tab_hbm, out_hbm in (
            (bidx_v, btab_hbm, bout_hbm),
            (pidx_v, ptab_hbm, pout_hbm),
        ):
            copies = []
            for c in range(_BPW // _CH):
                sl = pl.ds(c * _CH, _CH)
                copies.append(
                    pltpu.async_copy(tab_hbm.at[idx_v.at[sl]], rows_v.at[sl], sem))
            for cp in copies:
                cp.wait()
            pltpu.sync_copy(rows_v, out_hbm.at[pl.ds(base, _BPW)])

    return k(batter_idx, pitcher_idx, bquad, pquad)


def _sel_quarter(x, q_ref):
    q = q_ref[...]
    lo = jnp.where(q < 1.5,
                   jnp.where(q < 0.5, x[:, 0 * NQ:1 * NQ], x[:, 1 * NQ:2 * NQ]),
                   jnp.where(q < 2.5, x[:, 2 * NQ:3 * NQ], x[:, 3 * NQ:4 * NQ]))
    return lo


def _tc_body(bsel_ref, psel_ref, bq_ref, pq_ref, bias_ref, out_ref):
    y = (_sel_quarter(bsel_ref[...], bq_ref)
         + _sel_quarter(psel_ref[...], pq_ref) + bias_ref[...])
    out_ref[...] = y[:, :NOUT]


def _tc_dense(bsel, psel, bq, pq, fc_b):
    bias = jnp.pad(fc_b, (0, NQ - NOUT)).reshape(1, NQ)
    blk = 2048
    grid = (BATCH // blk,)
    return pl.pallas_call(
        _tc_body,
        grid=grid,
        in_specs=[
            pl.BlockSpec((blk, 4 * NQ), lambda i: (i, 0)),
            pl.BlockSpec((blk, 4 * NQ), lambda i: (i, 0)),
            pl.BlockSpec((blk, 1), lambda i: (i, 0)),
            pl.BlockSpec((blk, 1), lambda i: (i, 0)),
            pl.BlockSpec((1, NQ), lambda i: (0, 0)),
        ],
        out_specs=pl.BlockSpec((blk, NOUT), lambda i: (i, 0)),
        out_shape=jax.ShapeDtypeStruct((BATCH, NOUT), jnp.float32),
    )(bsel, psel, bq, pq, bias)


@jax.jit
def kernel(batter_idx, pitcher_idx, batter_table, pitcher_table, fc_w, fc_b):
    w1 = jnp.pad(fc_w[:, :VEC], ((0, NQ - NOUT), (0, 0)))
    w2 = jnp.pad(fc_w[:, VEC:], ((0, NQ - NOUT), (0, 0)))
    bquad = _tc_quad(batter_table.T, HB, w1)
    pquad = _tc_quad(pitcher_table.T, HP, w2)
    bsel, psel = _sc_gather(batter_idx, pitcher_idx, bquad, pquad)
    bq = (batter_idx // HB).astype(jnp.float32).reshape(-1, 1)
    pq = (pitcher_idx // HP).astype(jnp.float32).reshape(-1, 1)
    return _tc_dense(bsel, psel, bq, pq, fc_b)


# speedup vs baseline: 1.5196x; 1.0081x over previous
import functools

import jax
import jax.numpy as jnp
from jax import lax
from jax.experimental import pallas as pl
from jax.experimental.pallas import tpu as pltpu
from jax.experimental.pallas import tpu_sc as plsc

BATCH = 16384
VEC = 64
NOUT = 30
NQ = 32
HB = 262144
HP = 32768

_NC = 2
_NS = 16
_NW = _NC * _NS
_BPW = BATCH // _NW
_CH = 128


def _quad_body(x0_ref, x1_ref, x2_ref, x3_ref, wt_ref, out_ref):
    dn = (((1,), (0,)), ((), ()))
    wt = wt_ref[...]
    ys = []
    for x_ref in (x0_ref, x1_ref, x2_ref, x3_ref):
        s = jax.nn.sigmoid(x_ref[...])
        ys.append(lax.dot_general(wt, s, dn, preferred_element_type=jnp.float32))
    out_ref[...] = jnp.concatenate(ys, axis=0).T


def _tc_quad(tab_t, half, w_t):
    blk = 2048
    grid = (half // blk,)
    off = half // blk
    last = (tab_t.shape[1] + blk - 1) // blk - 1
    mk = lambda q: (lambda i: (0, jnp.minimum(i + q * off, last)))
    return pl.pallas_call(
        _quad_body,
        grid=grid,
        in_specs=[
            pl.BlockSpec((VEC, blk), mk(0)),
            pl.BlockSpec((VEC, blk), mk(1)),
            pl.BlockSpec((VEC, blk), mk(2)),
            pl.BlockSpec((VEC, blk), mk(3)),
            pl.BlockSpec((NQ, VEC), lambda i: (0, 0)),
        ],
        out_specs=pl.BlockSpec((blk, 4 * NQ), lambda i: (i, 0)),
        out_shape=jax.ShapeDtypeStruct((half, 4 * NQ), jnp.float32),
        compiler_params=pltpu.CompilerParams(fuse_transposed_lhs_in_matmul=True),
    )(tab_t, tab_t, tab_t, tab_t, w_t)


def _sc_gather(idx, quad, hmask):
    mesh = plsc.VectorSubcoreMesh(core_axis_name="c", subcore_axis_name="s")

    @functools.partial(
        pl.kernel,
        mesh=mesh,
        out_type=jax.ShapeDtypeStruct((BATCH, 4 * NQ), jnp.float32),
        scratch_types=[
            pltpu.VMEM((_BPW,), jnp.int32),
            pltpu.VMEM((_BPW, 4 * NQ), jnp.float32),
            pltpu.SemaphoreType.DMA,
        ],
    )
    def k(idx_hbm, tab_hbm, out_hbm, idx_v, rows_v, sem):
        wid = lax.axis_index("s") * _NC + lax.axis_index("c")
        base = wid * _BPW
        pltpu.sync_copy(idx_hbm.at[pl.ds(base, _BPW)], idx_v)
        for g in range(_BPW // 16):
            sl = pl.ds(g * 16, 16)
            idx_v[sl] = lax.bitwise_and(idx_v[sl], hmask)
        copies = []
        for c in range(_BPW // _CH):
            sl = pl.ds(c * _CH, _CH)
            copies.append(
                pltpu.async_copy(tab_hbm.at[idx_v.at[sl]], rows_v.at[sl], sem))
        for cp in copies:
            cp.wait()
        pltpu.sync_copy(rows_v, out_hbm.at[pl.ds(base, _BPW)])

    return k(idx, quad)


def _sel_quarter(x, q_ref):
    q = q_ref[...]
    lo = jnp.where(q < 1.5,
                   jnp.where(q < 0.5, x[:, 0 * NQ:1 * NQ], x[:, 1 * NQ:2 * NQ]),
                   jnp.where(q < 2.5, x[:, 2 * NQ:3 * NQ], x[:, 3 * NQ:4 * NQ]))
    return lo


def _tc_body(bsel_ref, psel_ref, bq_ref, pq_ref, bias_ref, out_ref):
    y = (_sel_quarter(bsel_ref[...], bq_ref)
         + _sel_quarter(psel_ref[...], pq_ref) + bias_ref[...])
    out_ref[...] = y[:, :NOUT]


def _tc_dense(bsel, psel, bq, pq, fc_b):
    bias = jnp.pad(fc_b, (0, NQ - NOUT)).reshape(1, NQ)
    blk = 2048
    grid = (BATCH // blk,)
    return pl.pallas_call(
        _tc_body,
        grid=grid,
        in_specs=[
            pl.BlockSpec((blk, 4 * NQ), lambda i: (i, 0)),
            pl.BlockSpec((blk, 4 * NQ), lambda i: (i, 0)),
            pl.BlockSpec((blk, 1), lambda i: (i, 0)),
            pl.BlockSpec((blk, 1), lambda i: (i, 0)),
            pl.BlockSpec((1, NQ), lambda i: (0, 0)),
        ],
        out_specs=pl.BlockSpec((blk, NOUT), lambda i: (i, 0)),
        out_shape=jax.ShapeDtypeStruct((BATCH, NOUT), jnp.float32),
    )(bsel, psel, bq, pq, bias)


@jax.jit
def kernel(batter_idx, pitcher_idx, batter_table, pitcher_table, fc_w, fc_b):
    w1 = jnp.pad(fc_w[:, :VEC], ((0, NQ - NOUT), (0, 0)))
    w2 = jnp.pad(fc_w[:, VEC:], ((0, NQ - NOUT), (0, 0)))
    pquad = _tc_quad(pitcher_table.T, HP, w2)
    psel = _sc_gather(pitcher_idx, pquad, HP - 1)
    bquad = _tc_quad(batter_table.T, HB, w1)
    bsel = _sc_gather(batter_idx, bquad, HB - 1)
    bq = (batter_idx // HB).astype(jnp.float32).reshape(-1, 1)
    pq = (pitcher_idx // HP).astype(jnp.float32).reshape(-1, 1)
    return _tc_dense(bsel, psel, bq, pq, fc_b)
